# baseline (speedup 1.0000x reference)
import functools

import jax
import jax.numpy as jnp
from jax import lax
from jax.experimental import pallas as pl
from jax.experimental.pallas import tpu as pltpu
from jax.experimental.pallas import tpu_sc as plsc

CHUNK = 128


def kernel(focuses, mask, pe):
    B, L = focuses.shape
    V, H = pe.shape
    N = B * L
    info = plsc.get_sparse_core_info()
    nc, ns = info.num_cores, info.num_subcores
    nw = nc * ns
    per_w = N // nw
    n_chunks = per_w // CHUNK
    assert per_w * nw == N and n_chunks * CHUNK == per_w

    zero_row = V
    pe_ext = jnp.concatenate([pe, jnp.zeros((1, H), jnp.float32)], axis=0)
    foc3 = focuses.reshape(nw, n_chunks, CHUNK)
    msk3 = mask.astype(jnp.int32).reshape(nw, n_chunks, CHUNK)

    @functools.partial(
        pl.kernel,
        mesh=plsc.VectorSubcoreMesh(core_axis_name="c", subcore_axis_name="s"),
        compiler_params=pltpu.CompilerParams(use_tc_tiling_on_sc=False),
        out_type=jax.ShapeDtypeStruct((N, H), jnp.float32),
        scratch_types=[
            pltpu.VMEM((n_chunks, CHUNK), jnp.int32),
            pltpu.VMEM((n_chunks, CHUNK), jnp.int32),
            pltpu.VMEM((CHUNK, H), jnp.float32),
            pltpu.SemaphoreType.DMA,
        ],
    )
    def fe_kernel(pe_hbm, foc_hbm, msk_hbm, out_hbm, idx_v, msk_v, rows_v, sem):
        c = lax.axis_index("c")
        s = lax.axis_index("s")
        wid = s * nc + c
        base = wid * per_w
        pltpu.sync_copy(foc_hbm.at[wid], idx_v)
        pltpu.sync_copy(msk_hbm.at[wid], msk_v)

        def fix(t, carry):
            j = t // (CHUNK // 16)
            k = (t % (CHUNK // 16)) * 16
            m = msk_v[j, pl.ds(k, 16)]
            f = idx_v[j, pl.ds(k, 16)]
            idx_v[j, pl.ds(k, 16)] = jnp.where(m > 0, f, zero_row)
            return carry

        lax.fori_loop(0, n_chunks * (CHUNK // 16), fix, 0)

        def chunk_body(g, carry):
            pltpu.async_copy(pe_hbm.at[idx_v.at[g]], rows_v, sem).wait()
            pltpu.sync_copy(rows_v, out_hbm.at[pl.ds(base + g * CHUNK, CHUNK)])
            return carry

        lax.fori_loop(0, n_chunks, chunk_body, 0)

    out = fe_kernel(pe_ext, foc3, msk3)
    return out.reshape(B, L, H)


# speedup vs baseline: 12.2939x; 12.2939x over previous
import functools

import jax
import jax.numpy as jnp
from jax import lax
from jax.experimental import pallas as pl
from jax.experimental.pallas import tpu as pltpu
from jax.experimental.pallas import tpu_sc as plsc

CHUNK = 128
NBUF = 4
LEAD = 2


def kernel(focuses, mask, pe):
    B, L = focuses.shape
    V, H = pe.shape
    N = B * L
    info = plsc.get_sparse_core_info()
    nc, ns = info.num_cores, info.num_subcores
    nw = nc * ns
    per_w = N // nw
    n_chunks = per_w // CHUNK
    n_outer = n_chunks // NBUF
    assert per_w * nw == N and n_outer * NBUF == n_chunks

    zero_row = V
    pe_ext = jnp.concatenate([pe, jnp.zeros((1, H), jnp.float32)], axis=0)
    foc3 = focuses.reshape(nw, n_chunks, CHUNK)
    msk3 = mask.astype(jnp.int32).reshape(nw, n_chunks, CHUNK)

    @functools.partial(
        pl.kernel,
        mesh=plsc.VectorSubcoreMesh(core_axis_name="c", subcore_axis_name="s"),
        compiler_params=pltpu.CompilerParams(use_tc_tiling_on_sc=False),
        out_type=jax.ShapeDtypeStruct((N, H), jnp.float32),
        scratch_types=[
            pltpu.VMEM((n_chunks, CHUNK), jnp.int32),
            pltpu.VMEM((n_chunks, CHUNK), jnp.int32),
            pltpu.VMEM((NBUF, CHUNK, H), jnp.float32),
            pltpu.VMEM_SHARED((V + 1, H), jnp.float32),
        ]
        + [pltpu.SemaphoreType.DMA] * (2 * NBUF),
    )
    def fe_kernel(pe_hbm, foc_hbm, msk_hbm, out_hbm, idx_v, msk_v, rows_v, pe_sh, *sems):
        gsems = sems[:NBUF]
        wsems = sems[NBUF:]
        c = lax.axis_index("c")
        s = lax.axis_index("s")
        wid = s * nc + c
        base = wid * per_w

        @pl.when(s == 0)
        def _():
            pltpu.sync_copy(pe_hbm, pe_sh)

        pltpu.sync_copy(foc_hbm.at[wid], idx_v)
        pltpu.sync_copy(msk_hbm.at[wid], msk_v)
        plsc.subcore_barrier()

        def outer(o, carry):
            for b in range(NBUF):
                g = o * NBUF + b

                @pl.when(g >= NBUF)
                def _():
                    pltpu.make_async_copy(
                        rows_v.at[b],
                        out_hbm.at[pl.ds(base + (g - NBUF) * CHUNK, CHUNK)],
                        wsems[b],
                    ).wait()

                for k in range(CHUNK // 16):
                    m = msk_v[g, pl.ds(k * 16, 16)]
                    f = idx_v[g, pl.ds(k * 16, 16)]
                    idx_v[g, pl.ds(k * 16, 16)] = jnp.where(m > 0, f, zero_row)

                pltpu.async_copy(pe_sh.at[idx_v.at[g]], rows_v.at[b], gsems[b])

                @pl.when(g >= LEAD)
                def _():
                    b2 = (b - LEAD) % NBUF
                    g2 = g - LEAD
                    pltpu.make_async_copy(
                        pe_sh.at[idx_v.at[g2]], rows_v.at[b2], gsems[b2]
                    ).wait()
                    pltpu.async_copy(
                        rows_v.at[b2],
                        out_hbm.at[pl.ds(base + g2 * CHUNK, CHUNK)],
                        wsems[b2],
                    )

            return carry

        lax.fori_loop(0, n_outer, outer, 0)

        for t in range(LEAD):
            g2 = n_chunks - LEAD + t
            b2 = g2 % NBUF
            pltpu.make_async_copy(
                pe_sh.at[idx_v.at[g2]], rows_v.at[b2], gsems[b2]
            ).wait()
            pltpu.async_copy(
                rows_v.at[b2],
                out_hbm.at[pl.ds(base + g2 * CHUNK, CHUNK)],
                wsems[b2],
            )
        for b in range(NBUF):
            g2 = n_chunks - NBUF + b
            pltpu.make_async_copy(
                rows_v.at[b],
                out_hbm.at[pl.ds(base + g2 * CHUNK, CHUNK)],
                wsems[b],
            ).wait()

    out = fe_kernel(pe_ext, foc3, msk3)
    return out.reshape(B, L, H)
